# Initial kernel scaffold; baseline (speedup 1.0000x reference)
import jax
import jax.numpy as jnp
from jax.experimental import pallas as pl


def kernel(src, emb_weight, pe):
    raise NotImplementedError("write your pallas kernel here")


# speedup vs baseline: 1.9966x; 1.9966x over previous
import functools
import math

import jax
import jax.numpy as jnp
from jax import lax
from jax.experimental import pallas as pl
from jax.experimental.pallas import tpu as pltpu
from jax.experimental.pallas import tpu_sc as plsc

_NTOKEN = 100000
_NINP = 1024
_SEQ = 4096
_BATCH = 4
_NROWS = _SEQ * _BATCH

_NC = 2
_NS = 16
_NW = _NC * _NS
_ROWS_PER_W = _NROWS // _NW
_CHUNK = 64
_NCHUNK = _ROWS_PER_W // _CHUNK


def _sc_gather_body(table_hbm, idx_hbm, out_hbm, idx_v, rows_v, sem):
    wid = lax.axis_index("s") * _NC + lax.axis_index("c")
    base = wid * _ROWS_PER_W
    for c in range(_NCHUNK):
        off = base + c * _CHUNK
        pltpu.sync_copy(idx_hbm.at[pl.ds(off, _CHUNK)], idx_v)
        pltpu.async_copy(table_hbm.at[idx_v], rows_v, sem).wait()
        pltpu.sync_copy(rows_v, out_hbm.at[pl.ds(off, _CHUNK)])


@functools.cache
def _sc_gather():
    return pl.kernel(
        _sc_gather_body,
        mesh=plsc.VectorSubcoreMesh(core_axis_name="c", subcore_axis_name="s"),
        out_type=jax.ShapeDtypeStruct((_NROWS, _NINP), jnp.float32),
        scratch_types=[
            pltpu.VMEM((_CHUNK,), jnp.int32),
            pltpu.VMEM((_CHUNK, _NINP), jnp.float32),
            pltpu.SemaphoreType.DMA,
        ],
    )


def _logsoftmax_body(g_ref, pe_ref, o_ref):
    y = g_ref[...] * math.sqrt(_NINP) + pe_ref[...]
    m = jnp.max(y, axis=-1, keepdims=True)
    e = jnp.exp(y - m)
    s = jnp.sum(e, axis=-1, keepdims=True)
    o_ref[...] = y - m - jnp.log(s)


_S_BLK = 256


def _tc_logsoftmax(g, pe):
    grid = (_SEQ // _S_BLK,)
    return pl.pallas_call(
        _logsoftmax_body,
        grid=grid,
        in_specs=[
            pl.BlockSpec((_S_BLK, _BATCH, _NINP), lambda i: (i, 0, 0)),
            pl.BlockSpec((_S_BLK, 1, _NINP), lambda i: (i, 0, 0)),
        ],
        out_specs=pl.BlockSpec((_S_BLK, _BATCH, _NINP), lambda i: (i, 0, 0)),
        out_shape=jax.ShapeDtypeStruct((_SEQ, _BATCH, _NINP), jnp.float32),
    )(g, pe)


def kernel(src, emb_weight, pe):
    idx = src.reshape(-1).astype(jnp.int32)
    gathered = _sc_gather()(emb_weight, idx)
    g = gathered.reshape(_SEQ, _BATCH, _NINP)
    return _tc_logsoftmax(g, pe)


# speedup vs baseline: 2.7267x; 1.3657x over previous
import functools
import math

import jax
import jax.numpy as jnp
from jax import lax
from jax.experimental import pallas as pl
from jax.experimental.pallas import tpu as pltpu
from jax.experimental.pallas import tpu_sc as plsc

_NTOKEN = 100000
_NINP = 1024
_SEQ = 4096
_BATCH = 4
_NROWS = _SEQ * _BATCH

_NC = 2
_NS = 16
_NW = _NC * _NS
_ROWS_PER_W = _NROWS // _NW
_CHUNK = 64
_NCHUNK = _ROWS_PER_W // _CHUNK


def _sc_gather_body(table_hbm, idx_hbm, out_hbm, idx_v, rows_v, sem):
    wid = lax.axis_index("s") * _NC + lax.axis_index("c")
    base = wid * _ROWS_PER_W
    for c in range(_NCHUNK):
        off = base + c * _CHUNK
        pltpu.sync_copy(idx_hbm.at[pl.ds(off, _CHUNK)], idx_v)
        pltpu.async_copy(table_hbm.at[idx_v], rows_v, sem).wait()
        pltpu.sync_copy(rows_v, out_hbm.at[pl.ds(off, _CHUNK)])


@functools.cache
def _sc_gather():
    return pl.kernel(
        _sc_gather_body,
        mesh=plsc.VectorSubcoreMesh(core_axis_name="c", subcore_axis_name="s"),
        out_type=jax.ShapeDtypeStruct((_NROWS, _NINP), jnp.float32),
        scratch_types=[
            pltpu.VMEM((_CHUNK,), jnp.int32),
            pltpu.VMEM((_CHUNK, _NINP), jnp.float32),
            pltpu.SemaphoreType.DMA,
        ],
    )


def _logsoftmax_body(g_ref, pe_ref, o_ref):
    pe2 = pe_ref[...]
    pe_exp = jnp.broadcast_to(
        pe2[:, None, :], (_S_BLK, _BATCH, _NINP)
    ).reshape(_S_BLK * _BATCH, _NINP)
    y = g_ref[...] * math.sqrt(_NINP) + pe_exp
    m = jnp.max(y, axis=-1, keepdims=True)
    e = jnp.exp(y - m)
    s = jnp.sum(e, axis=-1, keepdims=True)
    out2 = y - m - jnp.log(s)
    o_ref[...] = out2.reshape(_S_BLK, _BATCH, _NINP)


_S_BLK = 256


def _tc_logsoftmax(g2, pe2):
    grid = (_SEQ // _S_BLK,)
    return pl.pallas_call(
        _logsoftmax_body,
        grid=grid,
        in_specs=[
            pl.BlockSpec((_S_BLK * _BATCH, _NINP), lambda i: (i, 0)),
            pl.BlockSpec((_S_BLK, _NINP), lambda i: (i, 0)),
        ],
        out_specs=pl.BlockSpec((_S_BLK, _BATCH, _NINP), lambda i: (i, 0, 0)),
        out_shape=jax.ShapeDtypeStruct((_SEQ, _BATCH, _NINP), jnp.float32),
    )(g2, pe2)


def kernel(src, emb_weight, pe):
    idx = src.reshape(-1).astype(jnp.int32)
    gathered = _sc_gather()(emb_weight, idx)
    pe2 = pe.reshape(pe.shape[0], _NINP)
    return _tc_logsoftmax(gathered, pe2)


# speedup vs baseline: 3.2336x; 1.1859x over previous
import functools
import math

import jax
import jax.numpy as jnp
from jax import lax
from jax.experimental import pallas as pl
from jax.experimental.pallas import tpu as pltpu
from jax.experimental.pallas import tpu_sc as plsc

_NTOKEN = 100000
_NINP = 1024
_SEQ = 4096
_BATCH = 4
_NROWS = _SEQ * _BATCH

_NC = 2
_NS = 16
_NW = _NC * _NS
_ROWS_PER_W = _NROWS // _NW
_CHUNK = 64
_NCHUNK = _ROWS_PER_W // _CHUNK


def _sc_gather_body(table_hbm, idx_hbm, out_hbm, idx_v, rows_v, sem):
    wid = lax.axis_index("s") * _NC + lax.axis_index("c")
    base = wid * _ROWS_PER_W
    for c in range(_NCHUNK):
        off = base + c * _CHUNK
        pltpu.sync_copy(idx_hbm.at[pl.ds(off, _CHUNK)], idx_v)
        pltpu.async_copy(table_hbm.at[idx_v], rows_v, sem).wait()
        pltpu.sync_copy(rows_v, out_hbm.at[pl.ds(off, _CHUNK)])


@functools.cache
def _sc_gather():
    return pl.kernel(
        _sc_gather_body,
        mesh=plsc.VectorSubcoreMesh(core_axis_name="c", subcore_axis_name="s"),
        out_type=jax.ShapeDtypeStruct((_NROWS, _NINP), jnp.float32),
        scratch_types=[
            pltpu.VMEM((_CHUNK,), jnp.int32),
            pltpu.VMEM((_CHUNK, _NINP), jnp.float32),
            pltpu.SemaphoreType.DMA,
        ],
    )


def _logsoftmax_body(g_ref, pe_ref, o_ref):
    pe3 = pe_ref[...]
    pe_exp = jnp.broadcast_to(
        pe3, (_S_BLK, _BATCH, _NINP)
    ).reshape(_S_BLK * _BATCH, _NINP)
    y = g_ref[...] * math.sqrt(_NINP) + pe_exp
    m = jnp.max(y, axis=-1, keepdims=True)
    e = jnp.exp(y - m)
    s = jnp.sum(e, axis=-1, keepdims=True)
    out2 = y - m - jnp.log(s)
    o_ref[...] = out2.reshape(_S_BLK, _BATCH, _NINP)


_S_BLK = 256


def _tc_logsoftmax(g2, pe3):
    grid = (_SEQ // _S_BLK,)
    return pl.pallas_call(
        _logsoftmax_body,
        grid=grid,
        in_specs=[
            pl.BlockSpec((_S_BLK * _BATCH, _NINP), lambda i: (i, 0)),
            pl.BlockSpec((_S_BLK, 1, _NINP), lambda i: (i, 0, 0)),
        ],
        out_specs=pl.BlockSpec((_S_BLK, _BATCH, _NINP), lambda i: (i, 0, 0)),
        out_shape=jax.ShapeDtypeStruct((_SEQ, _BATCH, _NINP), jnp.float32),
    )(g2, pe3)


def kernel(src, emb_weight, pe):
    idx = src.reshape(-1).astype(jnp.int32)
    gathered = _sc_gather()(emb_weight, idx)
    return _tc_logsoftmax(gathered, pe)
